# baseline (speedup 1.0000x reference)
import dataclasses
import functools

import jax
import jax.numpy as jnp
import numpy as np
from jax import lax
from jax.experimental import pallas as pl
from jax.experimental.pallas import tpu as pltpu
from jax.experimental.pallas import tpu_sc as plsc

N = 10000
E = 320000
F = 128
H = 4
D = 32

NC = 2
NS = 16
NW = NC * NS
EPW = E // NW
G = 16
NGROUPS = EPW // G
NGP = 648
NTH = NGP // 3
ZCH = 16
NP = N + ZCH
NCHUNK = NP // ZCH
NB = 3
W144 = H * D + 16

_LEAKY = 0.2
_EPS = 1e-16


def _proj_body(x_ref, w_ref, cat_ref):
    cat_ref[...] = lax.dot_general(
        x_ref[...], w_ref[0], (((1,), (1,)), ((), ())),
        preferred_element_type=jnp.float32)


def _final_body(acc_ref, x_ref, bias_ref, m_ref, pa_ref, o_ref):
    acc = acc_ref[0] + acc_ref[1]
    num = acc[:, :H * D]
    den = acc[:, H * D:]
    denb = lax.dot_general(
        den, m_ref[...], (((1,), (0,)), ((), ())),
        preferred_element_type=jnp.float32)
    o = num / (denb + _EPS) + x_ref[...] + bias_ref[...]
    pa = pa_ref[...]
    o_ref[...] = jnp.where(o >= 0, o, pa * o)


def _gat_edge_kernel(cat_hbm, gidx_hbm, a_hbm, acc_out,
                     gidx_buf, rows0, rows1, rows2, msg0, msg1, msg2, pbuf,
                     abuf, zbuf, acc_acc,
                     gr0, gr1, gr2, sc0, sc1, sc2):
    cid = lax.axis_index("c")
    sid = lax.axis_index("s")
    wid = cid * NS + sid

    rows = [rows0, rows1, rows2]
    msg = [msg0, msg1, msg2]
    gr = [gr0, gr1, gr2]
    sc = [sc0, sc1, sc2]

    fzero = jnp.zeros((16,), jnp.float32)
    iota = lax.iota(jnp.int32, 16)
    ibase = iota * 16

    @pl.loop(0, ZCH)
    def _(r):
        for v in range(W144 // 16):
            zbuf[r, pl.ds(v * 16, 16)] = fzero

    @pl.loop(0, NCHUNK)
    def _(c):
        @pl.when(c % NS == sid)
        def _():
            pltpu.sync_copy(zbuf, acc_acc.at[pl.ds(c * ZCH, ZCH)])

    for b in range(NB):
        for j in range(G):
            msg[b][j, pl.ds(H * D, 16)] = fzero

    pltpu.sync_copy(a_hbm, abuf)
    av = [abuf[pl.ds(v * 16, 16)] for v in range(8)]
    cvec = [jnp.full((16,), j, jnp.int32) for j in range(G)]

    gdn = jax.lax.GatherDimensionNumbers(
        offset_dims=(), collapsed_slice_dims=(0,), start_index_map=(0,))

    plsc.subcore_barrier()

    def compute_group(b):
        for j in range(G):
            z = [rows[b][16 + j, pl.ds(v * 16, 16)]
                 + rows[b][j, pl.ds(v * 16, 16)] for v in range(8)]
            lk = [jnp.maximum(zv, _LEAKY * zv) for zv in z]
            t = [lk[v] * av[v] for v in range(8)]
            for h in range(H):
                plsc.store_scatter(pbuf, [ibase + (h * 256 + j)],
                                   t[2 * h] + t[2 * h + 1])

        attns = []
        for h in range(H):
            vals = [pbuf[pl.ds(h * 256 + l * 16, 16)] for l in range(16)]
            while len(vals) > 1:
                vals = [vals[i] + vals[i + 1] for i in range(0, len(vals), 2)]
            attn = jnp.exp(vals[0])
            attns.append(attn)
            plsc.store_scatter(msg[b], [iota, jnp.full((16,), H * D + h,
                                                       jnp.int32)], attn)

        for j in range(G):
            bc = [lax.gather(attns[h], cvec[j][:, None], gdn, (1,),
                             mode=lax.GatherScatterMode.PROMISE_IN_BOUNDS)
                  for h in range(H)]
            for v in range(8):
                msg[b][j, pl.ds(v * 16, 16)] = (
                    rows[b][16 + j, pl.ds(v * 16, 16)] * bc[v // 2])

    @pl.loop(0, 3)
    def _(third):
        t0 = pl.multiple_of(third * NTH, 8)
        pltpu.sync_copy(gidx_hbm.at[wid, pl.ds(t0, NTH)], gidx_buf)

        for b in range(NB):
            pltpu.async_copy(cat_hbm.at[gidx_buf.at[b]], rows[b], gr[b])

        @pl.loop(0, NTH, step=NB)
        def _(g):
            for b in range(NB):
                gb = g + b

                @pl.when(gb >= NB)
                def _():
                    od = gidx_buf[gb - NB, pl.ds(0, 16)]
                    pltpu.make_async_copy(msg[b], acc_acc.at[od], sc[b]).wait()

                pltpu.make_async_copy(
                    cat_hbm.at[gidx_buf.at[gb]], rows[b], gr[b]).wait()

                compute_group(b)

                didx_vec = gidx_buf[gb, pl.ds(0, 16)]
                pltpu.async_copy(msg[b], acc_acc.at[didx_vec], sc[b], add=True)

                @pl.when(gb + NB < NTH)
                def _():
                    pltpu.async_copy(
                        cat_hbm.at[gidx_buf.at[gb + NB]], rows[b], gr[b])

        for b in range(NB):
            od = gidx_buf[NTH - NB + b, pl.ds(0, 16)]
            pltpu.make_async_copy(msg[b], acc_acc.at[od], sc[b]).wait()

    plsc.subcore_barrier()

    @pl.loop(0, NCHUNK)
    def _(c):
        @pl.when(c % NS == sid)
        def _():
            pltpu.sync_copy(acc_acc.at[pl.ds(c * ZCH, ZCH)], zbuf)
            pltpu.sync_copy(zbuf, acc_out.at[cid, pl.ds(c * ZCH, ZCH)])


_HEAD_BCAST = np.zeros((16, 128), np.float32)
for _h in range(H):
    _HEAD_BCAST[_h, _h * D:(_h + 1) * D] = 1.0


@jax.jit
def kernel(x, edge_index, W_src, W_dst, double_attn, bias, prelu_a):
    npad = NGP - NGROUPS
    src2d = jnp.concatenate(
        [edge_index[0].reshape(NW, NGROUPS, 16) + N,
         jnp.full((NW, npad, 16), N, jnp.int32)], axis=1)
    dst2d = jnp.concatenate(
        [edge_index[1].reshape(NW, NGROUPS, 16),
         jnp.full((NW, npad, 16), N, jnp.int32)], axis=1)
    gidx = jnp.concatenate([dst2d, src2d], axis=2)
    a_flat = double_attn.reshape(H * D)
    w_cat = jnp.stack([W_dst, W_src])

    PB = 400
    cat = pl.pallas_call(
        _proj_body,
        grid=(2, N // PB),
        in_specs=[
            pl.BlockSpec((PB, F), lambda j, i: (i, 0)),
            pl.BlockSpec((1, H * D, F), lambda j, i: (j, 0, 0)),
        ],
        out_specs=pl.BlockSpec((PB, H * D), lambda j, i: (j * (N // PB) + i, 0)),
        out_shape=jax.ShapeDtypeStruct((2 * N, H * D), jnp.float32),
    )(x, w_cat)

    mesh = plsc.VectorSubcoreMesh(core_axis_name="c", subcore_axis_name="s")
    cp = pltpu.CompilerParams()
    if "needs_layout_passes" in pltpu.CompilerParams.__dataclass_fields__:
        cp = dataclasses.replace(cp, needs_layout_passes=False)
    if "use_tc_tiling_on_sc" in pltpu.CompilerParams.__dataclass_fields__:
        cp = dataclasses.replace(cp, use_tc_tiling_on_sc=False)
    sc_kernel = functools.partial(
        pl.kernel,
        compiler_params=cp,
        out_type=jax.ShapeDtypeStruct((NC, NP, W144), jnp.float32),
        mesh=mesh,
        scratch_types=[
            pltpu.VMEM((NTH, 32), jnp.int32),
            pltpu.VMEM((2 * G, H * D), jnp.float32),
            pltpu.VMEM((2 * G, H * D), jnp.float32),
            pltpu.VMEM((2 * G, H * D), jnp.float32),
            pltpu.VMEM((G, W144), jnp.float32),
            pltpu.VMEM((G, W144), jnp.float32),
            pltpu.VMEM((G, W144), jnp.float32),
            pltpu.VMEM((H * 256,), jnp.float32),
            pltpu.VMEM((H * D,), jnp.float32),
            pltpu.VMEM((ZCH, W144), jnp.float32),
            pltpu.VMEM_SHARED((NP, W144), jnp.float32),
            pltpu.SemaphoreType.DMA,
            pltpu.SemaphoreType.DMA,
            pltpu.SemaphoreType.DMA,
            pltpu.SemaphoreType.DMA,
            pltpu.SemaphoreType.DMA,
            pltpu.SemaphoreType.DMA,
        ],
    )(_gat_edge_kernel)
    acc_part = sc_kernel(cat, gidx, a_flat)
    acc_part = acc_part[:, :N]

    bias_row = bias.reshape(1, H * D)
    pa_row = jnp.broadcast_to(prelu_a.reshape(1, 1), (1, H * D))
    m = jnp.asarray(_HEAD_BCAST)
    out = pl.pallas_call(
        _final_body,
        grid=(N // PB,),
        in_specs=[
            pl.BlockSpec((NC, PB, W144), lambda i: (0, i, 0)),
            pl.BlockSpec((PB, F), lambda i: (i, 0)),
            pl.BlockSpec((1, H * D), lambda i: (0, 0)),
            pl.BlockSpec((16, H * D), lambda i: (0, 0)),
            pl.BlockSpec((1, H * D), lambda i: (0, 0)),
        ],
        out_specs=pl.BlockSpec((PB, H * D), lambda i: (i, 0)),
        out_shape=jax.ShapeDtypeStruct((N, H * D), jnp.float32),
    )(acc_part, x, bias_row, m, pa_row)
    return out


# speedup vs baseline: 2.5959x; 2.5959x over previous
import dataclasses
import functools

import jax
import jax.numpy as jnp
import numpy as np
from jax import lax
from jax.experimental import pallas as pl
from jax.experimental.pallas import tpu as pltpu
from jax.experimental.pallas import tpu_sc as plsc

N = 10000
E = 320000
F = 128
H = 4
D = 32

NC = 2
NS = 16
NW = NC * NS
EPW = E // NW
G = 16
NGROUPS = EPW // G
NGP = NGROUPS + 1
ZCH = 16
NP = N + ZCH
NCHUNK = NP // ZCH
NB = 2
W144 = H * D + 16

_LEAKY = 0.2
_EPS = 1e-16


def _proj_body(x_ref, w_ref, cat_ref, catb_ref):
    p = lax.dot_general(
        x_ref[...], w_ref[0], (((1,), (1,)), ((), ())),
        preferred_element_type=jnp.float32)
    cat_ref[...] = p
    catb_ref[...] = p.astype(jnp.bfloat16)


def _final_body(acc_ref, x_ref, bias_ref, m_ref, pa_ref, o_ref):
    acc = acc_ref[0] + acc_ref[1]
    num = acc[:, :H * D]
    den = acc[:, H * D:]
    denb = lax.dot_general(
        den, m_ref[...], (((1,), (0,)), ((), ())),
        preferred_element_type=jnp.float32)
    o = num / (denb + _EPS) + x_ref[...] + bias_ref[...]
    pa = pa_ref[...]
    o_ref[...] = jnp.where(o >= 0, o, pa * o)


def _gat_edge_kernel(cat_hbm, catb_hbm, gidx_hbm, a_hbm, acc_out,
                     gidx_buf, brows0, brows1, srows0, srows1,
                     msg0, msg1, pbuf, abuf, zbuf, acc_acc,
                     gr0, gr1, sc0, sc1):
    cid = lax.axis_index("c")
    sid = lax.axis_index("s")
    wid = cid * NS + sid

    brows = [brows0, brows1]
    srows = [srows0, srows1]
    msg = [msg0, msg1]
    gr = [gr0, gr1]
    sc = [sc0, sc1]

    fzero = jnp.zeros((16,), jnp.float32)
    iota = lax.iota(jnp.int32, 16)
    ibase = iota * 16

    @pl.loop(0, ZCH)
    def _(r):
        for v in range(W144 // 16):
            zbuf[r, pl.ds(v * 16, 16)] = fzero

    @pl.loop(0, NCHUNK)
    def _(c):
        @pl.when(c % NS == sid)
        def _():
            pltpu.sync_copy(zbuf, acc_acc.at[pl.ds(c * ZCH, ZCH)])

    for b in range(NB):
        for j in range(G):
            msg[b][j, pl.ds(H * D, 16)] = fzero

    pltpu.sync_copy(gidx_hbm.at[wid], gidx_buf)
    pltpu.sync_copy(a_hbm, abuf)
    av = [abuf[pl.ds(v * 32, 32)] for v in range(4)]
    cvec = [jnp.full((16,), j, jnp.int32) for j in range(G)]

    gdn = jax.lax.GatherDimensionNumbers(
        offset_dims=(), collapsed_slice_dims=(0,), start_index_map=(0,))

    plsc.subcore_barrier()

    def compute_group(b):
        for j in range(G):
            z = [brows[b][16 + j, pl.ds(v * 32, 32)]
                 + brows[b][j, pl.ds(v * 32, 32)] for v in range(4)]
            lk = [jnp.maximum(zv, jnp.bfloat16(_LEAKY) * zv) for zv in z]
            t = [lk[v] * av[v] for v in range(4)]
            for h in range(H):
                te, to = plsc.unpack(t[h], format=plsc.PackFormat.INTERLEAVED)
                plsc.store_scatter(pbuf, [ibase + (h * 256 + j)], te + to)

        attns = []
        for h in range(H):
            vals = [pbuf[pl.ds(h * 256 + l * 16, 16)] for l in range(16)]
            while len(vals) > 1:
                vals = [vals[i] + vals[i + 1] for i in range(0, len(vals), 2)]
            attn = jnp.exp(vals[0])
            attns.append(attn)
            plsc.store_scatter(msg[b], [iota, jnp.full((16,), H * D + h,
                                                       jnp.int32)], attn)

        for j in range(G):
            bc = [lax.gather(attns[h], cvec[j][:, None], gdn, (1,),
                             mode=lax.GatherScatterMode.PROMISE_IN_BOUNDS)
                  for h in range(H)]
            for v in range(8):
                msg[b][j, pl.ds(v * 16, 16)] = (
                    srows[b][j, pl.ds(v * 16, 16)] * bc[v // 2])

    def issue_gathers(gb, b):
        pltpu.async_copy(catb_hbm.at[gidx_buf.at[gb]], brows[b], gr[b])
        sidx = gidx_buf[gb, pl.ds(16, 16)]
        pltpu.async_copy(cat_hbm.at[sidx], srows[b], gr[b])

    def wait_gathers(gb, b):
        pltpu.make_async_copy(
            catb_hbm.at[gidx_buf.at[gb]], brows[b], gr[b]).wait()
        sidx = gidx_buf[gb, pl.ds(16, 16)]
        pltpu.make_async_copy(cat_hbm.at[sidx], srows[b], gr[b]).wait()

    for b in range(NB):
        issue_gathers(b, b)

    @pl.loop(0, NGP, step=NB)
    def _(g):
        for b in range(NB):
            gb = g + b

            @pl.when(gb >= NB)
            def _():
                od = gidx_buf[gb - NB, pl.ds(0, 16)]
                pltpu.make_async_copy(msg[b], acc_acc.at[od], sc[b]).wait()

            wait_gathers(gb, b)

            compute_group(b)

            didx_vec = gidx_buf[gb, pl.ds(0, 16)]
            pltpu.async_copy(msg[b], acc_acc.at[didx_vec], sc[b], add=True)

            @pl.when(gb + NB < NGP)
            def _():
                issue_gathers(gb + NB, b)

    for b in range(NB):
        od = gidx_buf[NGP - NB + b, pl.ds(0, 16)]
        pltpu.make_async_copy(msg[b], acc_acc.at[od], sc[b]).wait()

    plsc.subcore_barrier()

    @pl.loop(0, NCHUNK)
    def _(c):
        @pl.when(c % NS == sid)
        def _():
            pltpu.sync_copy(acc_acc.at[pl.ds(c * ZCH, ZCH)], zbuf)
            pltpu.sync_copy(zbuf, acc_out.at[cid, pl.ds(c * ZCH, ZCH)])


_HEAD_BCAST = np.zeros((16, 128), np.float32)
for _h in range(H):
    _HEAD_BCAST[_h, _h * D:(_h + 1) * D] = 1.0


@jax.jit
def kernel(x, edge_index, W_src, W_dst, double_attn, bias, prelu_a):
    src2d = jnp.concatenate(
        [edge_index[0].reshape(NW, NGROUPS, 16) + N,
         jnp.full((NW, 1, 16), N, jnp.int32)], axis=1)
    dst2d = jnp.concatenate(
        [edge_index[1].reshape(NW, NGROUPS, 16),
         jnp.full((NW, 1, 16), N, jnp.int32)], axis=1)
    gidx = jnp.concatenate([dst2d, src2d], axis=2)
    a_flat = double_attn.reshape(H * D)
    w_cat = jnp.stack([W_dst, W_src])

    PB = 400
    cat, catb = pl.pallas_call(
        _proj_body,
        grid=(2, N // PB),
        in_specs=[
            pl.BlockSpec((PB, F), lambda j, i: (i, 0)),
            pl.BlockSpec((1, H * D, F), lambda j, i: (j, 0, 0)),
        ],
        out_specs=[
            pl.BlockSpec((PB, H * D), lambda j, i: (j * (N // PB) + i, 0)),
            pl.BlockSpec((PB, H * D), lambda j, i: (j * (N // PB) + i, 0)),
        ],
        out_shape=[
            jax.ShapeDtypeStruct((2 * N, H * D), jnp.float32),
            jax.ShapeDtypeStruct((2 * N, H * D), jnp.bfloat16),
        ],
    )(x, w_cat)

    mesh = plsc.VectorSubcoreMesh(core_axis_name="c", subcore_axis_name="s")
    cp = pltpu.CompilerParams()
    if "needs_layout_passes" in pltpu.CompilerParams.__dataclass_fields__:
        cp = dataclasses.replace(cp, needs_layout_passes=False)
    if "use_tc_tiling_on_sc" in pltpu.CompilerParams.__dataclass_fields__:
        cp = dataclasses.replace(cp, use_tc_tiling_on_sc=False)
    sc_kernel = functools.partial(
        pl.kernel,
        compiler_params=cp,
        out_type=jax.ShapeDtypeStruct((NC, NP, W144), jnp.float32),
        mesh=mesh,
        scratch_types=[
            pltpu.VMEM((NGP, 32), jnp.int32),
            pltpu.VMEM((2 * G, H * D), jnp.bfloat16),
            pltpu.VMEM((2 * G, H * D), jnp.bfloat16),
            pltpu.VMEM((G, H * D), jnp.float32),
            pltpu.VMEM((G, H * D), jnp.float32),
            pltpu.VMEM((G, W144), jnp.float32),
            pltpu.VMEM((G, W144), jnp.float32),
            pltpu.VMEM((H * 256,), jnp.float32),
            pltpu.VMEM((H * D,), jnp.bfloat16),
            pltpu.VMEM((ZCH, W144), jnp.float32),
            pltpu.VMEM_SHARED((NP, W144), jnp.float32),
            pltpu.SemaphoreType.DMA,
            pltpu.SemaphoreType.DMA,
            pltpu.SemaphoreType.DMA,
            pltpu.SemaphoreType.DMA,
        ],
    )(_gat_edge_kernel)
    acc_part = sc_kernel(cat, catb, gidx, a_flat.astype(jnp.bfloat16))
    acc_part = acc_part[:, :N]

    bias_row = bias.reshape(1, H * D)
    pa_row = jnp.broadcast_to(prelu_a.reshape(1, 1), (1, H * D))
    m = jnp.asarray(_HEAD_BCAST)
    out = pl.pallas_call(
        _final_body,
        grid=(N // PB,),
        in_specs=[
            pl.BlockSpec((NC, PB, W144), lambda i: (0, i, 0)),
            pl.BlockSpec((PB, F), lambda i: (i, 0)),
            pl.BlockSpec((1, H * D), lambda i: (0, 0)),
            pl.BlockSpec((16, H * D), lambda i: (0, 0)),
            pl.BlockSpec((1, H * D), lambda i: (0, 0)),
        ],
        out_specs=pl.BlockSpec((PB, H * D), lambda i: (i, 0)),
        out_shape=jax.ShapeDtypeStruct((N, H * D), jnp.float32),
    )(acc_part, x, bias_row, m, pa_row)
    return out
